# Initial kernel scaffold; baseline (speedup 1.0000x reference)
import jax
import jax.numpy as jnp
from jax.experimental import pallas as pl


def kernel(x, W1, b1, W2, b2, W3, b3, Wg1, bg1, Wg2, bg2):
    raise NotImplementedError("write your pallas kernel here")


# speedup vs baseline: 1.6013x; 1.6013x over previous
import functools

import jax
import jax.numpy as jnp
from jax.experimental import pallas as pl
from jax.experimental.pallas import tpu as pltpu


def _gating_kernel(x_ref, wg1_ref, bg1_ref, wg2_ref, bg2_ref, gates_ref):
    x = x_ref[...].astype(jnp.bfloat16)
    h = jnp.dot(x, wg1_ref[...].astype(jnp.bfloat16),
                preferred_element_type=jnp.float32)
    h = jnp.maximum(h + bg1_ref[...], 0.0)
    logits = jnp.dot(h.astype(jnp.bfloat16), wg2_ref[...].astype(jnp.bfloat16),
                     preferred_element_type=jnp.float32) + bg2_ref[...]
    e_num = logits.shape[-1]
    iota = jax.lax.broadcasted_iota(jnp.int32, logits.shape, 1)
    m0 = jnp.max(logits, axis=1, keepdims=True)
    e0 = jnp.min(jnp.where(logits == m0, iota, e_num), axis=1, keepdims=True)
    l2 = jnp.where(iota == e0, -jnp.inf, logits)
    m1 = jnp.max(l2, axis=1, keepdims=True)
    e1 = jnp.min(jnp.where(l2 == m1, iota, e_num), axis=1, keepdims=True)
    t = jnp.exp(m1 - m0)
    w0 = 1.0 / (1.0 + t)
    w1 = t / (1.0 + t)
    gates_ref[...] = (jnp.where(iota == e0, w0, 0.0)
                      + jnp.where(iota == e1, w1, 0.0))


def _moe_dense_kernel(x_ref, w1_ref, b1_ref, w2_ref, b2_ref, w3_ref, b3_ref,
                      g_ref, o_ref):
    e = pl.program_id(1)
    x = x_ref[...]
    xb = x.astype(jnp.bfloat16)
    h = jnp.dot(xb, w1_ref[0], preferred_element_type=jnp.float32)
    h = jnp.maximum(h + b1_ref[0], 0.0)
    h = jnp.dot(h.astype(jnp.bfloat16), w2_ref[0],
                preferred_element_type=jnp.float32) + b2_ref[0]
    h = jnp.maximum(h + x, 0.0)
    y = jnp.dot(h.astype(jnp.bfloat16), w3_ref[0],
                preferred_element_type=jnp.float32) + b3_ref[0]
    col = jax.lax.broadcasted_iota(jnp.int32, g_ref.shape, 1)
    g = jnp.sum(jnp.where(col == e, g_ref[...], 0.0), axis=1, keepdims=True)
    acc = g * y

    @pl.when(e == 0)
    def _():
        o_ref[...] = acc

    @pl.when(e > 0)
    def _():
        o_ref[...] += acc


def kernel(x, W1, b1, W2, b2, W3, b3, Wg1, bg1, Wg2, bg2):
    B, D = x.shape
    E, _, H = W1.shape
    O = W3.shape[-1]
    G = Wg1.shape[-1]

    gt = min(B, 1024)
    gates = pl.pallas_call(
        _gating_kernel,
        grid=(B // gt,),
        in_specs=[
            pl.BlockSpec((gt, D), lambda i: (i, 0)),
            pl.BlockSpec((D, G), lambda i: (0, 0)),
            pl.BlockSpec((1, G), lambda i: (0, 0)),
            pl.BlockSpec((G, E), lambda i: (0, 0)),
            pl.BlockSpec((1, E), lambda i: (0, 0)),
        ],
        out_specs=pl.BlockSpec((gt, E), lambda i: (i, 0)),
        out_shape=jax.ShapeDtypeStruct((B, E), jnp.float32),
    )(x, Wg1, bg1.reshape(1, G), Wg2, bg2.reshape(1, E))

    w1b = W1.astype(jnp.bfloat16)
    w2b = W2.astype(jnp.bfloat16)
    w3b = W3.astype(jnp.bfloat16)

    tb = min(B, 1024)
    out = pl.pallas_call(
        _moe_dense_kernel,
        grid=(B // tb, E),
        in_specs=[
            pl.BlockSpec((tb, D), lambda i, e: (i, 0)),
            pl.BlockSpec((1, D, H), lambda i, e: (e, 0, 0)),
            pl.BlockSpec((1, 1, H), lambda i, e: (e, 0, 0)),
            pl.BlockSpec((1, H, H), lambda i, e: (e, 0, 0)),
            pl.BlockSpec((1, 1, H), lambda i, e: (e, 0, 0)),
            pl.BlockSpec((1, H, O), lambda i, e: (e, 0, 0)),
            pl.BlockSpec((1, 1, O), lambda i, e: (e, 0, 0)),
            pl.BlockSpec((tb, E), lambda i, e: (i, 0)),
        ],
        out_specs=pl.BlockSpec((tb, O), lambda i, e: (i, 0)),
        out_shape=jax.ShapeDtypeStruct((B, O), jnp.float32),
        compiler_params=pltpu.CompilerParams(
            dimension_semantics=("arbitrary", "arbitrary"),
        ),
    )(x, w1b, b1.reshape(E, 1, H), w2b, b2.reshape(E, 1, H),
      w3b, b3.reshape(E, 1, O), gates)
    return out
